# baseline (speedup 1.0000x reference)
import functools

import jax
import jax.numpy as jnp
from jax import lax
from jax.experimental import pallas as pl
from jax.experimental.pallas import tpu as pltpu
from jax.experimental.pallas import tpu_sc as plsc

F = 26
V = 100000
D = 32
B = 16384
R = F * D

NC = 2
NS = 16
L = 16
NW = NC * NS
IPW = R // NW
CHUNK = 2048
NCHUNK = B // CHUNK
UNROLL = 4

_mesh = plsc.VectorSubcoreMesh(core_axis_name="c", subcore_axis_name="s")


@functools.partial(
    pl.kernel,
    out_type=jax.ShapeDtypeStruct((R, B), jnp.float32),
    mesh=_mesh,
    scratch_types=[
        pltpu.VMEM((V,), jnp.float32),
        pltpu.VMEM((B,), jnp.int32),
        pltpu.VMEM((CHUNK,), jnp.float32),
        pltpu.VMEM((CHUNK,), jnp.float32),
        pltpu.SemaphoreType.DMA,
        pltpu.SemaphoreType.DMA,
        pltpu.SemaphoreType.DMA,
    ],
    compiler_params=pltpu.CompilerParams(
        use_tc_tiling_on_sc=True, needs_layout_passes=False
    ),
)
def _gather_kernel(tt, xt, out, row_v, idx_v, val0_v, val1_v, sem0, sem1, rsem):
    wid = lax.axis_index("s") * NC + lax.axis_index("c")
    vals = (val0_v, val1_v)
    sems = (sem0, sem1)

    def item_body(k, prev_f):
        t = wid * IPW + k
        f = t // D
        j = t % D

        h0 = pltpu.async_copy(tt.at[f, j, :], row_v, rsem)

        @pl.when(f != prev_f)
        def _():
            pltpu.sync_copy(xt.at[f, :], idx_v)

        h0.wait()

        handles = []
        for c in range(NCHUNK):
            val_v = vals[c % 2]
            if c >= 2:
                handles[c - 2].wait()

            def gather_body(i, carry3, _c=c, _val=val_v):
                base = _c * CHUNK + i * (L * UNROLL)
                for u in range(UNROLL):
                    s = pl.ds(base + u * L, L)
                    d = pl.ds(i * (L * UNROLL) + u * L, L)
                    _val[d] = plsc.load_gather(row_v, [idx_v[s]])
                return carry3

            lax.fori_loop(0, CHUNK // (L * UNROLL), gather_body, 0)
            handles.append(
                pltpu.async_copy(val_v, out.at[t, pl.ds(c * CHUNK, CHUNK)],
                                 sems[c % 2])
            )
        handles[NCHUNK - 2].wait()
        handles[NCHUNK - 1].wait()
        return f

    lax.fori_loop(0, IPW, item_body, -1)


def kernel(x, tables):
    if x.ndim <= 1:
        x = x[None, :]
    xt = x.T
    tt = jnp.transpose(tables, (0, 2, 1))
    out_t = _gather_kernel(tt, xt)
    return out_t.T


# speedup vs baseline: 1.9965x; 1.9965x over previous
import functools

import jax
import jax.numpy as jnp
from jax import lax
from jax.experimental import pallas as pl
from jax.experimental.pallas import tpu as pltpu
from jax.experimental.pallas import tpu_sc as plsc

F = 26
V = 100000
D = 32
B = 16384
R = F * D

NC = 2
NS = 16
L = 16
NW = NC * NS
IPW = R // NW
CHUNK = 2048
NCHUNK = B // CHUNK
UNROLL = 8

_mesh = plsc.VectorSubcoreMesh(core_axis_name="c", subcore_axis_name="s")


@functools.partial(
    pl.kernel,
    out_type=jax.ShapeDtypeStruct((R, B), jnp.float32),
    mesh=_mesh,
    scratch_types=[
        pltpu.VMEM((V,), jnp.float32),
        pltpu.VMEM((B,), jnp.int32),
        pltpu.VMEM((CHUNK,), jnp.float32),
        pltpu.VMEM((CHUNK,), jnp.float32),
        pltpu.SemaphoreType.DMA,
        pltpu.SemaphoreType.DMA,
        pltpu.SemaphoreType.DMA,
    ],
    compiler_params=pltpu.CompilerParams(
        use_tc_tiling_on_sc=True, needs_layout_passes=False
    ),
)
def _gather_kernel(tt, xt, out, row_v, idx_v, val0_v, val1_v, sem0, sem1, rsem):
    wid = lax.axis_index("s") * NC + lax.axis_index("c")
    vals = (val0_v, val1_v)
    sems = (sem0, sem1)

    def item_body(k, prev_f):
        t = wid * IPW + k
        f = t // D
        j = t % D

        h0 = pltpu.async_copy(tt.at[f, j, :], row_v, rsem)

        @pl.when(f != prev_f)
        def _():
            pltpu.sync_copy(xt.at[f, :], idx_v)

        h0.wait()

        handles = []
        for c in range(NCHUNK):
            val_v = vals[c % 2]
            if c >= 2:
                handles[c - 2].wait()

            def _gbody(i, _c=c, _val=val_v):
                _val[pl.ds(i, L)] = plsc.load_gather(
                    row_v, [idx_v[pl.ds(_c * CHUNK + i, L)]]
                )

            plsc.parallel_loop(0, CHUNK, step=L, unroll=UNROLL)(_gbody)
            handles.append(
                pltpu.async_copy(val_v, out.at[t, pl.ds(c * CHUNK, CHUNK)],
                                 sems[c % 2])
            )
        handles[NCHUNK - 2].wait()
        handles[NCHUNK - 1].wait()
        return f

    lax.fori_loop(0, IPW, item_body, -1)


def kernel(x, tables):
    if x.ndim <= 1:
        x = x[None, :]
    xt = x.T
    tt = jnp.transpose(tables, (0, 2, 1))
    out_t = _gather_kernel(tt, xt)
    return out_t.T
